# baseline (speedup 1.0000x reference)
import functools

import jax
import jax.numpy as jnp
from jax import lax
from jax.experimental import pallas as pl
from jax.experimental.pallas import tpu as pltpu

N = 10000
NPAD = 10240
SUB, LANE = 8, 1280
H, W = 32, 32
FX, FY, CX, CY = 30.0, 30.0, 16.0, 16.0

C0 = 0.28209479177387814
C1 = 0.4886025119029199
C2 = (1.0925484305920792, -1.0925484305920792, 0.31539156525252005,
      -1.0925484305920792, 0.5462742152960396)
C3 = (-0.5900435899266435, 2.890611442640554, -0.4570457994644658,
      0.3731763325901154, -0.4570457994644658, 1.445305721320277,
      -0.5900435899266435)

NATTR = 10
GCHUNK = 500
NGRID = N // GCHUNK
UNROLL = 10
LOG2E = 1.4426950408889634


def _bf(x):
    return x.astype(jnp.bfloat16).astype(jnp.float32)


def _stage_kernel(sc_ref, feat_ref, attrs_ref, areas_ref):
    f = lambda i: feat_ref[i]
    r = [sc_ref[i] for i in range(9)]
    t = [sc_ref[9 + i] for i in range(3)]
    br = [sc_ref[16 + i] for i in range(9)]

    px, py, pz = f(0), f(1), f(2)
    bpx, bpy, bpz = _bf(px), _bf(py), _bf(pz)
    pcx = bpx * br[0] + bpy * br[1] + bpz * br[2] + t[0]
    pcy = bpx * br[3] + bpy * br[4] + bpz * br[5] + t[1]
    pcz = bpx * br[6] + bpy * br[7] + bpz * br[8] + t[2]
    depth = pcz
    ux = FX * pcx / depth + CX
    uy = FY * pcy / depth + CY

    qw, qx, qy, qz = f(55), f(56), f(57), f(58)
    qn = jnp.sqrt(qw * qw + qx * qx + qy * qy + qz * qz) + 1e-12
    w, x, y, z = qw / qn, qx / qn, qy / qn, qz / qn
    R00 = 1 - 2 * (y * y + z * z)
    R01 = 2 * (x * y - w * z)
    R02 = 2 * (x * z + w * y)
    R10 = 2 * (x * y + w * z)
    R11 = 1 - 2 * (x * x + z * z)
    R12 = 2 * (y * z - w * x)
    R20 = 2 * (x * z - w * y)
    R21 = 2 * (y * z + w * x)
    R22 = 1 - 2 * (x * x + y * y)

    s0, s1, s2 = jnp.exp(f(52)), jnp.exp(f(53)), jnp.exp(f(54))
    M00, M01, M02 = _bf(R00 * s0), _bf(R01 * s1), _bf(R02 * s2)
    M10, M11, M12 = _bf(R10 * s0), _bf(R11 * s1), _bf(R12 * s2)
    M20, M21, M22 = _bf(R20 * s0), _bf(R21 * s1), _bf(R22 * s2)
    S00 = _bf(M00 * M00 + M01 * M01 + M02 * M02)
    S01 = _bf(M00 * M10 + M01 * M11 + M02 * M12)
    S02 = _bf(M00 * M20 + M01 * M21 + M02 * M22)
    S11 = _bf(M10 * M10 + M11 * M11 + M12 * M12)
    S12 = _bf(M10 * M20 + M11 * M21 + M12 * M22)
    S22 = _bf(M20 * M20 + M21 * M21 + M22 * M22)

    j00 = _bf(FX / depth)
    j02 = _bf(-FX * pcx / (depth * depth))
    j11 = _bf(FY / depth)
    j12 = _bf(-FY * pcy / (depth * depth))
    T0a = _bf(j00 * br[0] + j02 * br[6])
    T0b = _bf(j00 * br[1] + j02 * br[7])
    T0c = _bf(j00 * br[2] + j02 * br[8])
    T1a = _bf(j11 * br[3] + j12 * br[6])
    T1b = _bf(j11 * br[4] + j12 * br[7])
    T1c = _bf(j11 * br[5] + j12 * br[8])
    P0a = _bf(T0a * S00 + T0b * S01 + T0c * S02)
    P0b = _bf(T0a * S01 + T0b * S11 + T0c * S12)
    P0c = _bf(T0a * S02 + T0b * S12 + T0c * S22)
    P1a = _bf(T1a * S00 + T1b * S01 + T1c * S02)
    P1b = _bf(T1a * S01 + T1b * S11 + T1c * S12)
    P1c = _bf(T1a * S02 + T1b * S12 + T1c * S22)
    a = P0a * T0a + P0b * T0b + P0c * T0c + 0.3
    b = P0a * T1a + P0b * T1b + P0c * T1c
    c = P1a * T1a + P1b * T1b + P1c * T1c + 0.3

    det = a * c - b * b
    ca2 = (-0.5 * LOG2E) * (c / det)
    cb2 = -LOG2E * (-b / det)
    cc2 = (-0.5 * LOG2E) * (a / det)
    area_x = jnp.ceil(3.0 * jnp.sqrt(a))
    area_y = jnp.ceil(3.0 * jnp.sqrt(c))

    twx = -(r[0] * t[0] + r[3] * t[1] + r[6] * t[2])
    twy = -(r[1] * t[0] + r[4] * t[1] + r[7] * t[2])
    twz = -(r[2] * t[0] + r[5] * t[1] + r[8] * t[2])
    dx = px - twx
    dy = py - twy
    dz = pz - twz
    dn = jnp.sqrt(dx * dx + dy * dy + dz * dz) + 1e-12
    dx, dy, dz = dx / dn, dy / dn, dz / dn

    xx, yy, zz = dx * dx, dy * dy, dz * dz
    xy, yz, xz = dx * dy, dy * dz, dx * dz
    basis = [
        None,
        -C1 * dy, C1 * dz, -C1 * dx,
        C2[0] * xy, C2[1] * yz, C2[2] * (2 * zz - xx - yy),
        C2[3] * xz, C2[4] * (xx - yy),
        C3[0] * dy * (3 * xx - yy), C3[1] * xy * dz,
        C3[2] * dy * (4 * zz - xx - yy),
        C3[3] * dz * (2 * zz - 3 * xx - 3 * yy),
        C3[4] * dx * (4 * zz - xx - yy),
        C3[5] * dz * (xx - yy), C3[6] * dx * (xx - 3 * yy),
    ]
    cols = []
    for ch in range(3):
        sh = lambda k: feat_ref[3 + 3 * k + ch]
        col = C0 * sh(0)
        for k in range(1, 16):
            col = col + basis[k] * sh(k)
        cols.append(_bf(jnp.maximum(col + 0.5, 0.0)))

    alpha = 1.0 / (1.0 + jnp.exp(-f(51)))

    gid = (lax.broadcasted_iota(jnp.int32, (SUB, LANE), 0) * LANE
           + lax.broadcasted_iota(jnp.int32, (SUB, LANE), 1))
    pad = gid >= N
    depth = jnp.where(pad, jnp.inf, depth)
    alpha = jnp.where(pad, 0.0, alpha)

    attrs_ref[0] = ux
    attrs_ref[1] = uy
    attrs_ref[2] = ca2
    attrs_ref[3] = cb2
    attrs_ref[4] = cc2
    attrs_ref[5] = alpha
    attrs_ref[6] = cols[0]
    attrs_ref[7] = cols[1]
    attrs_ref[8] = cols[2]
    attrs_ref[9] = depth
    areas_ref[0] = area_x
    areas_ref[1] = area_y


def _splat_kernel(order_ref, attrs_ref, out_ref):
    p = (lax.broadcasted_iota(jnp.int32, (8, 128), 0) * 128
         + lax.broadcasted_iota(jnp.int32, (8, 128), 1))
    xf = (p % W).astype(jnp.float32)
    yf = (p // W).astype(jnp.float32)

    def body(gu, carry):
        T, aR, aG, aB = carry
        g0 = gu * UNROLL
        for k in range(UNROLL):
            g = order_ref[g0 + k]
            ux = attrs_ref[0, g]
            uy = attrs_ref[1, g]
            ca = attrs_ref[2, g]
            cb = attrs_ref[3, g]
            cc = attrs_ref[4, g]
            al = attrs_ref[5, g]
            cr = attrs_ref[6, g]
            cg = attrs_ref[7, g]
            cbl = attrs_ref[8, g]
            dx = xf - ux
            dy = yf - uy
            p2 = (ca * dx + cb * dy) * dx + (cc * dy) * dy
            gv = jnp.exp2(jnp.minimum(p2, 0.0))
            ae = jnp.minimum(al * gv, 0.99)
            ae = jnp.where(ae < (1.0 / 255.0), 0.0, ae)
            wgt = ae * T
            T = T - wgt
            aR = aR + wgt * cr
            aG = aG + wgt * cg
            aB = aB + wgt * cbl
        return (T, aR, aG, aB)

    ones = jnp.ones((8, 128), jnp.float32)
    zeros = jnp.zeros((8, 128), jnp.float32)
    T, aR, aG, aB = lax.fori_loop(
        0, N // UNROLL, body, (ones, zeros, zeros, zeros))
    out_ref[0] = aR
    out_ref[1] = aG
    out_ref[2] = aB


def kernel(pws, shs, alphas_raw, scales_raw, rots_raw, us, Rcw, tcw):
    del us
    feat = jnp.concatenate(
        [pws.T, shs.T, alphas_raw.T, scales_raw.T, rots_raw.T], axis=0)
    feat = jnp.pad(feat, ((0, 64 - feat.shape[0]), (0, NPAD - N)))
    feat = feat.reshape(64, SUB, LANE)
    sc = jnp.concatenate(
        [Rcw.reshape(9), tcw.reshape(3), jnp.zeros(4, jnp.float32),
         Rcw.reshape(9).astype(jnp.bfloat16).astype(jnp.float32),
         jnp.zeros(7, jnp.float32)])

    attrs, areas2 = pl.pallas_call(
        _stage_kernel,
        grid=(),
        in_specs=[
            pl.BlockSpec(memory_space=pltpu.SMEM),
            pl.BlockSpec(memory_space=pltpu.VMEM),
        ],
        out_specs=[
            pl.BlockSpec(memory_space=pltpu.VMEM),
            pl.BlockSpec(memory_space=pltpu.VMEM),
        ],
        out_shape=[
            jax.ShapeDtypeStruct((NATTR, SUB, LANE), jnp.float32),
            jax.ShapeDtypeStruct((2, SUB, LANE), jnp.float32),
        ],
    )(sc, feat)

    attrs_flat = attrs.reshape(NATTR, NPAD)
    order = jnp.argsort(attrs_flat[9]).astype(jnp.int32)
    img = pl.pallas_call(
        _splat_kernel,
        in_specs=[
            pl.BlockSpec(memory_space=pltpu.SMEM),
            pl.BlockSpec(memory_space=pltpu.SMEM),
        ],
        out_specs=pl.BlockSpec(memory_space=pltpu.VMEM),
        out_shape=jax.ShapeDtypeStruct((3, 8, 128), jnp.float32),
    )(order[:N], attrs_flat)

    image = img.reshape(3, H, W)
    areas = areas2.reshape(2, NPAD)[:, :N].T
    return image, areas


# speedup vs baseline: 235.0479x; 235.0479x over previous
import functools

import jax
import jax.numpy as jnp
from jax import lax
from jax.experimental import pallas as pl
from jax.experimental.pallas import tpu as pltpu

N = 10000
NPAD = 10240
SUB, LANE = 8, 1280
H, W = 32, 32
FX, FY, CX, CY = 30.0, 30.0, 16.0, 16.0

C0 = 0.28209479177387814
C1 = 0.4886025119029199
C2 = (1.0925484305920792, -1.0925484305920792, 0.31539156525252005,
      -1.0925484305920792, 0.5462742152960396)
C3 = (-0.5900435899266435, 2.890611442640554, -0.4570457994644658,
      0.3731763325901154, -0.4570457994644658, 1.445305721320277,
      -0.5900435899266435)

NATTR = 10
GCHUNK = 500
NGRID = N // GCHUNK
UNROLL = 10
LOG2E = 1.4426950408889634


def _bf(x):
    return x.astype(jnp.bfloat16).astype(jnp.float32)


def _stage_kernel(sc_ref, feat_ref, attrs_ref, areas_ref):
    f = lambda i: feat_ref[i]
    r = [sc_ref[i] for i in range(9)]
    t = [sc_ref[9 + i] for i in range(3)]
    br = [sc_ref[16 + i] for i in range(9)]

    px, py, pz = f(0), f(1), f(2)
    bpx, bpy, bpz = _bf(px), _bf(py), _bf(pz)
    pcx = bpx * br[0] + bpy * br[1] + bpz * br[2] + t[0]
    pcy = bpx * br[3] + bpy * br[4] + bpz * br[5] + t[1]
    pcz = bpx * br[6] + bpy * br[7] + bpz * br[8] + t[2]
    depth = pcz
    ux = FX * pcx / depth + CX
    uy = FY * pcy / depth + CY

    qw, qx, qy, qz = f(55), f(56), f(57), f(58)
    qn = jnp.sqrt(qw * qw + qx * qx + qy * qy + qz * qz) + 1e-12
    w, x, y, z = qw / qn, qx / qn, qy / qn, qz / qn
    R00 = 1 - 2 * (y * y + z * z)
    R01 = 2 * (x * y - w * z)
    R02 = 2 * (x * z + w * y)
    R10 = 2 * (x * y + w * z)
    R11 = 1 - 2 * (x * x + z * z)
    R12 = 2 * (y * z - w * x)
    R20 = 2 * (x * z - w * y)
    R21 = 2 * (y * z + w * x)
    R22 = 1 - 2 * (x * x + y * y)

    s0, s1, s2 = jnp.exp(f(52)), jnp.exp(f(53)), jnp.exp(f(54))
    M00, M01, M02 = _bf(R00 * s0), _bf(R01 * s1), _bf(R02 * s2)
    M10, M11, M12 = _bf(R10 * s0), _bf(R11 * s1), _bf(R12 * s2)
    M20, M21, M22 = _bf(R20 * s0), _bf(R21 * s1), _bf(R22 * s2)
    S00 = _bf(M00 * M00 + M01 * M01 + M02 * M02)
    S01 = _bf(M00 * M10 + M01 * M11 + M02 * M12)
    S02 = _bf(M00 * M20 + M01 * M21 + M02 * M22)
    S11 = _bf(M10 * M10 + M11 * M11 + M12 * M12)
    S12 = _bf(M10 * M20 + M11 * M21 + M12 * M22)
    S22 = _bf(M20 * M20 + M21 * M21 + M22 * M22)

    j00 = _bf(FX / depth)
    j02 = _bf(-FX * pcx / (depth * depth))
    j11 = _bf(FY / depth)
    j12 = _bf(-FY * pcy / (depth * depth))
    T0a = _bf(j00 * br[0] + j02 * br[6])
    T0b = _bf(j00 * br[1] + j02 * br[7])
    T0c = _bf(j00 * br[2] + j02 * br[8])
    T1a = _bf(j11 * br[3] + j12 * br[6])
    T1b = _bf(j11 * br[4] + j12 * br[7])
    T1c = _bf(j11 * br[5] + j12 * br[8])
    P0a = _bf(T0a * S00 + T0b * S01 + T0c * S02)
    P0b = _bf(T0a * S01 + T0b * S11 + T0c * S12)
    P0c = _bf(T0a * S02 + T0b * S12 + T0c * S22)
    P1a = _bf(T1a * S00 + T1b * S01 + T1c * S02)
    P1b = _bf(T1a * S01 + T1b * S11 + T1c * S12)
    P1c = _bf(T1a * S02 + T1b * S12 + T1c * S22)
    a = P0a * T0a + P0b * T0b + P0c * T0c + 0.3
    b = P0a * T1a + P0b * T1b + P0c * T1c
    c = P1a * T1a + P1b * T1b + P1c * T1c + 0.3

    det = a * c - b * b
    ca2 = (-0.5 * LOG2E) * (c / det)
    cb2 = -LOG2E * (-b / det)
    cc2 = (-0.5 * LOG2E) * (a / det)
    area_x = jnp.ceil(3.0 * jnp.sqrt(a))
    area_y = jnp.ceil(3.0 * jnp.sqrt(c))

    twx = -(r[0] * t[0] + r[3] * t[1] + r[6] * t[2])
    twy = -(r[1] * t[0] + r[4] * t[1] + r[7] * t[2])
    twz = -(r[2] * t[0] + r[5] * t[1] + r[8] * t[2])
    dx = px - twx
    dy = py - twy
    dz = pz - twz
    dn = jnp.sqrt(dx * dx + dy * dy + dz * dz) + 1e-12
    dx, dy, dz = dx / dn, dy / dn, dz / dn

    xx, yy, zz = dx * dx, dy * dy, dz * dz
    xy, yz, xz = dx * dy, dy * dz, dx * dz
    basis = [
        None,
        -C1 * dy, C1 * dz, -C1 * dx,
        C2[0] * xy, C2[1] * yz, C2[2] * (2 * zz - xx - yy),
        C2[3] * xz, C2[4] * (xx - yy),
        C3[0] * dy * (3 * xx - yy), C3[1] * xy * dz,
        C3[2] * dy * (4 * zz - xx - yy),
        C3[3] * dz * (2 * zz - 3 * xx - 3 * yy),
        C3[4] * dx * (4 * zz - xx - yy),
        C3[5] * dz * (xx - yy), C3[6] * dx * (xx - 3 * yy),
    ]
    cols = []
    for ch in range(3):
        sh = lambda k: feat_ref[3 + 3 * k + ch]
        col = C0 * sh(0)
        for k in range(1, 16):
            col = col + basis[k] * sh(k)
        cols.append(_bf(jnp.maximum(col + 0.5, 0.0)))

    alpha = 1.0 / (1.0 + jnp.exp(-f(51)))

    gid = (lax.broadcasted_iota(jnp.int32, (SUB, LANE), 0) * LANE
           + lax.broadcasted_iota(jnp.int32, (SUB, LANE), 1))
    pad = gid >= N
    depth = jnp.where(pad, jnp.inf, depth)
    alpha = jnp.where(pad, 0.0, alpha)

    attrs_ref[0] = ux
    attrs_ref[1] = uy
    attrs_ref[2] = ca2
    attrs_ref[3] = cb2
    attrs_ref[4] = cc2
    attrs_ref[5] = alpha
    attrs_ref[6] = cols[0]
    attrs_ref[7] = cols[1]
    attrs_ref[8] = cols[2]
    attrs_ref[9] = depth
    areas_ref[0] = area_x
    areas_ref[1] = area_y


def _splat_kernel(chunk_ref, out_ref, st_ref):
    i = pl.program_id(0)

    @pl.when(i == 0)
    def _():
        st_ref[0] = jnp.ones((8, 128), jnp.float32)
        st_ref[1] = jnp.zeros((8, 128), jnp.float32)
        st_ref[2] = jnp.zeros((8, 128), jnp.float32)
        st_ref[3] = jnp.zeros((8, 128), jnp.float32)

    p = (lax.broadcasted_iota(jnp.int32, (8, 128), 0) * 128
         + lax.broadcasted_iota(jnp.int32, (8, 128), 1))
    xf = (p % W).astype(jnp.float32)
    yf = (p // W).astype(jnp.float32)

    def body(gu, carry):
        T, aR, aG, aB = carry
        g0 = gu * UNROLL
        for k in range(UNROLL):
            g = g0 + k
            ux = chunk_ref[0, 0, g]
            uy = chunk_ref[0, 1, g]
            ca = chunk_ref[0, 2, g]
            cb = chunk_ref[0, 3, g]
            cc = chunk_ref[0, 4, g]
            al = chunk_ref[0, 5, g]
            cr = chunk_ref[0, 6, g]
            cg = chunk_ref[0, 7, g]
            cbl = chunk_ref[0, 8, g]
            dx = xf - ux
            dy = yf - uy
            p2 = (ca * dx + cb * dy) * dx + (cc * dy) * dy
            gv = jnp.exp2(jnp.minimum(p2, 0.0))
            ae = jnp.minimum(al * gv, 0.99)
            ae = jnp.where(ae < (1.0 / 255.0), 0.0, ae)
            wgt = ae * T
            T = T - wgt
            aR = aR + wgt * cr
            aG = aG + wgt * cg
            aB = aB + wgt * cbl
        return (T, aR, aG, aB)

    T, aR, aG, aB = lax.fori_loop(
        0, GCHUNK // UNROLL, body, (st_ref[0], st_ref[1], st_ref[2], st_ref[3]))
    st_ref[0] = T
    st_ref[1] = aR
    st_ref[2] = aG
    st_ref[3] = aB

    @pl.when(i == NGRID - 1)
    def _():
        out_ref[0] = aR
        out_ref[1] = aG
        out_ref[2] = aB


def kernel(pws, shs, alphas_raw, scales_raw, rots_raw, us, Rcw, tcw):
    del us
    feat = jnp.concatenate(
        [pws.T, shs.T, alphas_raw.T, scales_raw.T, rots_raw.T], axis=0)
    feat = jnp.pad(feat, ((0, 64 - feat.shape[0]), (0, NPAD - N)))
    feat = feat.reshape(64, SUB, LANE)
    sc = jnp.concatenate(
        [Rcw.reshape(9), tcw.reshape(3), jnp.zeros(4, jnp.float32),
         Rcw.reshape(9).astype(jnp.bfloat16).astype(jnp.float32),
         jnp.zeros(7, jnp.float32)])

    attrs, areas2 = pl.pallas_call(
        _stage_kernel,
        grid=(),
        in_specs=[
            pl.BlockSpec(memory_space=pltpu.SMEM),
            pl.BlockSpec(memory_space=pltpu.VMEM),
        ],
        out_specs=[
            pl.BlockSpec(memory_space=pltpu.VMEM),
            pl.BlockSpec(memory_space=pltpu.VMEM),
        ],
        out_shape=[
            jax.ShapeDtypeStruct((NATTR, SUB, LANE), jnp.float32),
            jax.ShapeDtypeStruct((2, SUB, LANE), jnp.float32),
        ],
    )(sc, feat)

    attrs_flat = attrs.reshape(NATTR, NPAD)
    order = jnp.argsort(attrs_flat[9]).astype(jnp.int32)
    chunks = (attrs_flat[:, order[:N]]
              .reshape(NATTR, NGRID, GCHUNK).transpose(1, 0, 2))

    img = pl.pallas_call(
        _splat_kernel,
        grid=(NGRID,),
        in_specs=[
            pl.BlockSpec((1, NATTR, GCHUNK), lambda i: (i, 0, 0),
                         memory_space=pltpu.SMEM),
        ],
        out_specs=pl.BlockSpec((3, 8, 128), lambda i: (0, 0, 0)),
        out_shape=jax.ShapeDtypeStruct((3, 8, 128), jnp.float32),
        scratch_shapes=[pltpu.VMEM((4, 8, 128), jnp.float32)],
    )(chunks)

    image = img.reshape(3, H, W)
    areas = areas2.reshape(2, NPAD)[:, :N].T
    return image, areas


# speedup vs baseline: 236.8163x; 1.0075x over previous
import functools

import jax
import jax.numpy as jnp
from jax import lax
from jax.experimental import pallas as pl
from jax.experimental.pallas import tpu as pltpu

N = 10000
NPAD = 10240
SUB, LANE = 8, 1280
H, W = 32, 32
FX, FY, CX, CY = 30.0, 30.0, 16.0, 16.0

C0 = 0.28209479177387814
C1 = 0.4886025119029199
C2 = (1.0925484305920792, -1.0925484305920792, 0.31539156525252005,
      -1.0925484305920792, 0.5462742152960396)
C3 = (-0.5900435899266435, 2.890611442640554, -0.4570457994644658,
      0.3731763325901154, -0.4570457994644658, 1.445305721320277,
      -0.5900435899266435)

NATTR = 10
DTAB = 16
GCHUNK = 512
NGRID = NPAD // GCHUNK
UNROLL = 8
LOG2E = 1.4426950408889634

def _bf(x):
    return x.astype(jnp.bfloat16).astype(jnp.float32)


def _stage_kernel(sc_ref, feat_ref, attrs_ref, areas_ref):
    f = lambda i: feat_ref[i]
    r = [sc_ref[i] for i in range(9)]
    t = [sc_ref[9 + i] for i in range(3)]
    br = [sc_ref[16 + i] for i in range(9)]

    px, py, pz = f(0), f(1), f(2)
    bpx, bpy, bpz = _bf(px), _bf(py), _bf(pz)
    pcx = bpx * br[0] + bpy * br[1] + bpz * br[2] + t[0]
    pcy = bpx * br[3] + bpy * br[4] + bpz * br[5] + t[1]
    pcz = bpx * br[6] + bpy * br[7] + bpz * br[8] + t[2]
    depth = pcz
    ux = FX * pcx / depth + CX
    uy = FY * pcy / depth + CY

    qw, qx, qy, qz = f(55), f(56), f(57), f(58)
    qn = jnp.sqrt(qw * qw + qx * qx + qy * qy + qz * qz) + 1e-12
    w, x, y, z = qw / qn, qx / qn, qy / qn, qz / qn
    R00 = 1 - 2 * (y * y + z * z)
    R01 = 2 * (x * y - w * z)
    R02 = 2 * (x * z + w * y)
    R10 = 2 * (x * y + w * z)
    R11 = 1 - 2 * (x * x + z * z)
    R12 = 2 * (y * z - w * x)
    R20 = 2 * (x * z - w * y)
    R21 = 2 * (y * z + w * x)
    R22 = 1 - 2 * (x * x + y * y)

    s0, s1, s2 = jnp.exp(f(52)), jnp.exp(f(53)), jnp.exp(f(54))
    M00, M01, M02 = _bf(R00 * s0), _bf(R01 * s1), _bf(R02 * s2)
    M10, M11, M12 = _bf(R10 * s0), _bf(R11 * s1), _bf(R12 * s2)
    M20, M21, M22 = _bf(R20 * s0), _bf(R21 * s1), _bf(R22 * s2)
    S00 = _bf(M00 * M00 + M01 * M01 + M02 * M02)
    S01 = _bf(M00 * M10 + M01 * M11 + M02 * M12)
    S02 = _bf(M00 * M20 + M01 * M21 + M02 * M22)
    S11 = _bf(M10 * M10 + M11 * M11 + M12 * M12)
    S12 = _bf(M10 * M20 + M11 * M21 + M12 * M22)
    S22 = _bf(M20 * M20 + M21 * M21 + M22 * M22)

    j00 = _bf(FX / depth)
    j02 = _bf(-FX * pcx / (depth * depth))
    j11 = _bf(FY / depth)
    j12 = _bf(-FY * pcy / (depth * depth))
    T0a = _bf(j00 * br[0] + j02 * br[6])
    T0b = _bf(j00 * br[1] + j02 * br[7])
    T0c = _bf(j00 * br[2] + j02 * br[8])
    T1a = _bf(j11 * br[3] + j12 * br[6])
    T1b = _bf(j11 * br[4] + j12 * br[7])
    T1c = _bf(j11 * br[5] + j12 * br[8])
    P0a = _bf(T0a * S00 + T0b * S01 + T0c * S02)
    P0b = _bf(T0a * S01 + T0b * S11 + T0c * S12)
    P0c = _bf(T0a * S02 + T0b * S12 + T0c * S22)
    P1a = _bf(T1a * S00 + T1b * S01 + T1c * S02)
    P1b = _bf(T1a * S01 + T1b * S11 + T1c * S12)
    P1c = _bf(T1a * S02 + T1b * S12 + T1c * S22)
    a = P0a * T0a + P0b * T0b + P0c * T0c + 0.3
    b = P0a * T1a + P0b * T1b + P0c * T1c
    c = P1a * T1a + P1b * T1b + P1c * T1c + 0.3

    det = a * c - b * b
    ca2 = (-0.5 * LOG2E) * (c / det)
    cb2 = -LOG2E * (-b / det)
    cc2 = (-0.5 * LOG2E) * (a / det)
    area_x = jnp.ceil(3.0 * jnp.sqrt(a))
    area_y = jnp.ceil(3.0 * jnp.sqrt(c))

    twx = -(r[0] * t[0] + r[3] * t[1] + r[6] * t[2])
    twy = -(r[1] * t[0] + r[4] * t[1] + r[7] * t[2])
    twz = -(r[2] * t[0] + r[5] * t[1] + r[8] * t[2])
    dx = px - twx
    dy = py - twy
    dz = pz - twz
    dn = jnp.sqrt(dx * dx + dy * dy + dz * dz) + 1e-12
    dx, dy, dz = dx / dn, dy / dn, dz / dn

    xx, yy, zz = dx * dx, dy * dy, dz * dz
    xy, yz, xz = dx * dy, dy * dz, dx * dz
    basis = [
        None,
        -C1 * dy, C1 * dz, -C1 * dx,
        C2[0] * xy, C2[1] * yz, C2[2] * (2 * zz - xx - yy),
        C2[3] * xz, C2[4] * (xx - yy),
        C3[0] * dy * (3 * xx - yy), C3[1] * xy * dz,
        C3[2] * dy * (4 * zz - xx - yy),
        C3[3] * dz * (2 * zz - 3 * xx - 3 * yy),
        C3[4] * dx * (4 * zz - xx - yy),
        C3[5] * dz * (xx - yy), C3[6] * dx * (xx - 3 * yy),
    ]
    cols = []
    for ch in range(3):
        sh = lambda k: feat_ref[3 + 3 * k + ch]
        col = C0 * sh(0)
        for k in range(1, 16):
            col = col + basis[k] * sh(k)
        cols.append(_bf(jnp.maximum(col + 0.5, 0.0)))

    alpha = 1.0 / (1.0 + jnp.exp(-f(51)))

    gid = (lax.broadcasted_iota(jnp.int32, (SUB, LANE), 0) * LANE
           + lax.broadcasted_iota(jnp.int32, (SUB, LANE), 1))
    pad = gid >= N
    depth = jnp.where(pad, jnp.inf, depth)
    alpha = jnp.where(pad, 0.0, alpha)

    attrs_ref[0] = ux
    attrs_ref[1] = uy
    attrs_ref[2] = ca2
    attrs_ref[3] = cb2
    attrs_ref[4] = cc2
    attrs_ref[5] = alpha
    attrs_ref[6] = cols[0]
    attrs_ref[7] = cols[1]
    attrs_ref[8] = cols[2]
    attrs_ref[9] = depth
    areas_ref[0] = area_x
    areas_ref[1] = area_y


def _splat_kernel(chunk_ref, out_ref, st_ref):
    i = pl.program_id(0)

    @pl.when(i == 0)
    def _():
        st_ref[0] = jnp.ones((8, 128), jnp.float32)
        st_ref[1] = jnp.zeros((8, 128), jnp.float32)
        st_ref[2] = jnp.zeros((8, 128), jnp.float32)
        st_ref[3] = jnp.zeros((8, 128), jnp.float32)

    p = (lax.broadcasted_iota(jnp.int32, (8, 128), 0) * 128
         + lax.broadcasted_iota(jnp.int32, (8, 128), 1))
    xf = (p % W).astype(jnp.float32)
    yf = (p // W).astype(jnp.float32)

    def body(gu, carry):
        T, aR, aG, aB = carry
        g0 = gu * UNROLL
        for k in range(UNROLL):
            g = g0 + k
            ux = chunk_ref[0, g]
            uy = chunk_ref[1, g]
            ca = chunk_ref[2, g]
            cb = chunk_ref[3, g]
            cc = chunk_ref[4, g]
            al = chunk_ref[5, g]
            cr = chunk_ref[6, g]
            cg = chunk_ref[7, g]
            cbl = chunk_ref[8, g]
            dx = xf - ux
            dy = yf - uy
            p2 = (ca * dx + cb * dy) * dx + (cc * dy) * dy
            gv = jnp.exp2(jnp.minimum(p2, 0.0))
            ae = jnp.minimum(al * gv, 0.99)
            ae = jnp.where(ae < (1.0 / 255.0), 0.0, ae)
            wgt = ae * T
            T = T - wgt
            aR = aR + wgt * cr
            aG = aG + wgt * cg
            aB = aB + wgt * cbl
        return (T, aR, aG, aB)

    T, aR, aG, aB = lax.fori_loop(
        0, GCHUNK // UNROLL, body, (st_ref[0], st_ref[1], st_ref[2], st_ref[3]))
    st_ref[0] = T
    st_ref[1] = aR
    st_ref[2] = aG
    st_ref[3] = aB

    @pl.when(i == NGRID - 1)
    def _():
        out_ref[0] = aR
        out_ref[1] = aG
        out_ref[2] = aB


def kernel(pws, shs, alphas_raw, scales_raw, rots_raw, us, Rcw, tcw):
    del us
    feat = jnp.concatenate(
        [pws.T, shs.T, alphas_raw.T, scales_raw.T, rots_raw.T], axis=0)
    feat = jnp.pad(feat, ((0, 64 - feat.shape[0]), (0, NPAD - N)))
    feat = feat.reshape(64, SUB, LANE)
    sc = jnp.concatenate(
        [Rcw.reshape(9), tcw.reshape(3), jnp.zeros(4, jnp.float32),
         Rcw.reshape(9).astype(jnp.bfloat16).astype(jnp.float32),
         jnp.zeros(7, jnp.float32)])

    attrs, areas2 = pl.pallas_call(
        _stage_kernel,
        grid=(),
        in_specs=[
            pl.BlockSpec(memory_space=pltpu.SMEM),
            pl.BlockSpec(memory_space=pltpu.VMEM),
        ],
        out_specs=[
            pl.BlockSpec(memory_space=pltpu.VMEM),
            pl.BlockSpec(memory_space=pltpu.VMEM),
        ],
        out_shape=[
            jax.ShapeDtypeStruct((NATTR, SUB, LANE), jnp.float32),
            jax.ShapeDtypeStruct((2, SUB, LANE), jnp.float32),
        ],
    )(sc, feat)

    attrs_flat = attrs.reshape(NATTR, NPAD)
    order = jnp.argsort(attrs_flat[9]).astype(jnp.int32)
    chunks = attrs_flat[:, order]

    img = pl.pallas_call(
        _splat_kernel,
        grid=(NGRID,),
        in_specs=[
            pl.BlockSpec((NATTR, GCHUNK), lambda i: (0, i),
                         memory_space=pltpu.SMEM),
        ],
        out_specs=pl.BlockSpec((3, 8, 128), lambda i: (0, 0, 0)),
        out_shape=jax.ShapeDtypeStruct((3, 8, 128), jnp.float32),
        scratch_shapes=[pltpu.VMEM((4, 8, 128), jnp.float32)],
    )(chunks)

    image = img.reshape(3, H, W)
    areas = areas2.reshape(2, NPAD)[:, :N].T
    return image, areas
